# baseline (speedup 1.0000x reference)
import functools

import jax
import jax.numpy as jnp
from jax import lax
from jax.experimental import pallas as pl
from jax.experimental.pallas import tpu as pltpu
from jax.experimental.pallas import tpu_sc as plsc

_TM = 512


def _dist_argmin_body(x2_ref, e2_ref, iota_ref, x_ref, emb_ref, idx_ref):
    xs = x_ref[...] * (-2.0)
    emb = emb_ref[...]
    nxe2 = lax.dot_general(
        xs, emb, (((1,), (1,)), ((), ())),
        preferred_element_type=jnp.float32)
    d = (x2_ref[...] + e2_ref[...]) + nxe2
    m = jnp.min(d, axis=1, keepdims=True)
    cand = jnp.where(d == m, iota_ref[...], jnp.float32(d.shape[1]))
    idxf = jnp.min(cand, axis=1, keepdims=True)
    idx_ref[...] = idxf.astype(jnp.int32)


def _nearest_code_indices(x2, e2, xf, embeddings):
    m, d = xf.shape
    k = embeddings.shape[0]
    iotaf = lax.iota(jnp.float32, k)[None]
    grid = (m // _TM,)
    return pl.pallas_call(
        _dist_argmin_body,
        grid=grid,
        in_specs=[
            pl.BlockSpec((_TM, 1), lambda i: (i, 0)),
            pl.BlockSpec((1, k), lambda i: (0, 0)),
            pl.BlockSpec((1, k), lambda i: (0, 0)),
            pl.BlockSpec((_TM, d), lambda i: (i, 0)),
            pl.BlockSpec((k, d), lambda i: (0, 0)),
        ],
        out_specs=pl.BlockSpec((_TM, 1), lambda i: (i, 0)),
        out_shape=jax.ShapeDtypeStruct((m, 1), jnp.int32),
        compiler_params=pltpu.CompilerParams(
            dimension_semantics=("parallel",)),
    )(x2, e2, iotaf, xf, embeddings)


def _gather_rows(table, idx):
    b = idx.shape[0]
    d = table.shape[1]
    info = plsc.get_sparse_core_info()
    nw = info.num_cores * info.num_subcores
    b_per_w = b // nw
    mesh = plsc.VectorSubcoreMesh(core_axis_name="c", subcore_axis_name="s")

    @functools.partial(
        pl.kernel, mesh=mesh,
        out_type=jax.ShapeDtypeStruct((b, d), jnp.float32),
        scratch_types=[
            pltpu.VMEM((b_per_w,), jnp.int32),
            pltpu.VMEM((b_per_w, d), jnp.float32),
            pltpu.SemaphoreType.DMA,
        ],
    )
    def gather_kernel(table_hbm, idx_hbm, out_hbm, idx_v, rows_v, sem):
        wid = lax.axis_index("s") * info.num_cores + lax.axis_index("c")
        base = wid * b_per_w
        pltpu.sync_copy(idx_hbm.at[pl.ds(base, b_per_w)], idx_v)
        pltpu.async_copy(table_hbm.at[idx_v], rows_v, sem).wait()
        pltpu.sync_copy(rows_v, out_hbm.at[pl.ds(base, b_per_w)])

    return gather_kernel(table, idx)


def kernel(x, embeddings):
    bsz, t, d = x.shape
    m = bsz * t
    xf = x.reshape(m, d)
    x2 = jnp.sum(xf * xf, axis=-1, keepdims=True)
    e2 = jnp.sum(embeddings * embeddings, axis=-1)[None]
    ind = _nearest_code_indices(x2, e2, xf, embeddings)
    emb = _gather_rows(embeddings, ind.reshape(m))
    emb = emb.reshape(bsz, t, d)
    return (emb, emb)


# speedup vs baseline: 1.1745x; 1.1745x over previous
import functools

import jax
import jax.numpy as jnp
from jax import lax
from jax.experimental import pallas as pl
from jax.experimental.pallas import tpu as pltpu
from jax.experimental.pallas import tpu_sc as plsc

_TM = 1024


def _dist_argmin_body(x2_ref, e2_ref, x_ref, emb_ref, idx_ref):
    x = x_ref[...]
    emb = emb_ref[...]
    xe = lax.dot_general(
        x, emb, (((1,), (1,)), ((), ())),
        preferred_element_type=jnp.float32)
    d = (x2_ref[...] + e2_ref[...]) - 2.0 * xe
    m = jnp.min(d, axis=1, keepdims=True)
    iot = lax.broadcasted_iota(jnp.int32, d.shape, 1).astype(jnp.float32)
    cand = jnp.where(d == m, iot, jnp.float32(d.shape[1]))
    idxf = jnp.min(cand, axis=1, keepdims=True)
    idx_ref[...] = idxf.astype(jnp.int32)


def _nearest_code_indices(x2, e2, xf, embeddings):
    m, d = xf.shape
    k = embeddings.shape[0]
    grid = (m // _TM,)
    return pl.pallas_call(
        _dist_argmin_body,
        grid=grid,
        in_specs=[
            pl.BlockSpec((_TM, 1), lambda i: (i, 0)),
            pl.BlockSpec((1, k), lambda i: (0, 0)),
            pl.BlockSpec((_TM, d), lambda i: (i, 0)),
            pl.BlockSpec((k, d), lambda i: (0, 0)),
        ],
        out_specs=pl.BlockSpec((_TM, 1), lambda i: (i, 0)),
        out_shape=jax.ShapeDtypeStruct((m, 1), jnp.int32),
        compiler_params=pltpu.CompilerParams(
            dimension_semantics=("parallel",)),
    )(x2, e2, xf, embeddings)


def _gather_rows(table, idx):
    b = idx.shape[0]
    d = table.shape[1]
    info = plsc.get_sparse_core_info()
    nw = info.num_cores * info.num_subcores
    b_per_w = b // nw
    mesh = plsc.VectorSubcoreMesh(core_axis_name="c", subcore_axis_name="s")

    @functools.partial(
        pl.kernel, mesh=mesh,
        out_type=jax.ShapeDtypeStruct((b, d), jnp.float32),
        scratch_types=[
            pltpu.VMEM((b_per_w,), jnp.int32),
            pltpu.VMEM((b_per_w, d), jnp.float32),
            pltpu.SemaphoreType.DMA,
        ],
    )
    def gather_kernel(table_hbm, idx_hbm, out_hbm, idx_v, rows_v, sem):
        wid = lax.axis_index("s") * info.num_cores + lax.axis_index("c")
        base = wid * b_per_w
        pltpu.sync_copy(idx_hbm.at[pl.ds(base, b_per_w)], idx_v)
        pltpu.async_copy(table_hbm.at[idx_v], rows_v, sem).wait()
        pltpu.sync_copy(rows_v, out_hbm.at[pl.ds(base, b_per_w)])

    return gather_kernel(table, idx)


def kernel(x, embeddings):
    bsz, t, d = x.shape
    m = bsz * t
    xf = x.reshape(m, d)
    x2 = jnp.sum(xf * xf, axis=-1, keepdims=True)
    e2 = jnp.sum(embeddings * embeddings, axis=-1)[None]
    ind = _nearest_code_indices(x2, e2, xf, embeddings)
    emb = _gather_rows(embeddings, ind.reshape(m))
    emb = emb.reshape(bsz, t, d)
    return (emb, emb)


# speedup vs baseline: 1.2160x; 1.0353x over previous
import functools

import jax
import jax.numpy as jnp
from jax import lax
from jax.experimental import pallas as pl
from jax.experimental.pallas import tpu as pltpu
from jax.experimental.pallas import tpu_sc as plsc

_TM = 1024


def _dist_argmin_body(x2_ref, e2_ref, x_ref, emb_ref, idx_ref):
    x = x_ref[0]
    emb = emb_ref[...]
    xe = lax.dot_general(
        x, emb, (((1,), (1,)), ((), ())),
        preferred_element_type=jnp.float32)
    d = (x2_ref[0] + e2_ref[...]) - 2.0 * xe
    m = jnp.min(d, axis=1, keepdims=True)
    iot = lax.broadcasted_iota(jnp.int32, d.shape, 1).astype(jnp.float32)
    cand = jnp.where(d == m, iot, jnp.float32(d.shape[1]))
    idxf = jnp.min(cand, axis=1, keepdims=True)
    idx_ref[0] = idxf.astype(jnp.int32)


def _nearest_code_indices(x2, e2, x, embeddings):
    bsz, t, d = x.shape
    k = embeddings.shape[0]
    grid = (bsz * t // _TM,)
    tb = _TM // t if _TM > t else 1
    return pl.pallas_call(
        _dist_argmin_body,
        grid=grid,
        in_specs=[
            pl.BlockSpec((tb, _TM // tb, 1), lambda i: (i, 0, 0)),
            pl.BlockSpec((1, k), lambda i: (0, 0)),
            pl.BlockSpec((tb, _TM // tb, d), lambda i: (i, 0, 0)),
            pl.BlockSpec((k, d), lambda i: (0, 0)),
        ],
        out_specs=pl.BlockSpec((tb, _TM // tb, 1), lambda i: (i, 0, 0)),
        out_shape=jax.ShapeDtypeStruct((bsz, t, 1), jnp.int32),
        compiler_params=pltpu.CompilerParams(
            dimension_semantics=("parallel",)),
    )(x2, e2, x, embeddings)


def _gather_rows(table, idx, bsz, t):
    b = idx.shape[0]
    d = table.shape[1]
    info = plsc.get_sparse_core_info()
    nw = info.num_cores * info.num_subcores
    b_per_w = b // nw
    t_per_w = t // b_per_w
    mesh = plsc.VectorSubcoreMesh(core_axis_name="c", subcore_axis_name="s")
    out_sd = jax.ShapeDtypeStruct((bsz, t, d), jnp.float32)

    @functools.partial(
        pl.kernel, mesh=mesh,
        out_type=(out_sd, out_sd),
        scratch_types=[
            pltpu.VMEM((b_per_w,), jnp.int32),
            pltpu.VMEM((b_per_w, d), jnp.float32),
            pltpu.SemaphoreType.DMA,
        ],
    )
    def gather_kernel(table_hbm, idx_hbm, out0_hbm, out1_hbm,
                      idx_v, rows_v, sem):
        wid = lax.axis_index("s") * info.num_cores + lax.axis_index("c")
        base = wid * b_per_w
        row = wid // t_per_w
        col = (wid % t_per_w) * b_per_w
        pltpu.sync_copy(idx_hbm.at[pl.ds(base, b_per_w)], idx_v)
        pltpu.async_copy(table_hbm.at[idx_v], rows_v, sem).wait()
        pltpu.sync_copy(rows_v, out0_hbm.at[row, pl.ds(col, b_per_w)])
        pltpu.sync_copy(rows_v, out1_hbm.at[row, pl.ds(col, b_per_w)])

    return gather_kernel(table, idx)


def kernel(x, embeddings):
    bsz, t, d = x.shape
    m = bsz * t
    x2 = jnp.sum(x * x, axis=-1, keepdims=True)
    e2 = jnp.sum(embeddings * embeddings, axis=-1)[None]
    ind = _nearest_code_indices(x2, e2, x, embeddings)
    out0, out1 = _gather_rows(embeddings, ind.reshape(m), bsz, t)
    return (out0, out1)
